# Initial kernel scaffold; baseline (speedup 1.0000x reference)
import jax
import jax.numpy as jnp
from jax.experimental import pallas as pl


def kernel(h, edge_index, e, snorm_n, snorm_e, W_e1, W_e2, W_ih, W_hh, ln_gamma, ln_beta):
    raise NotImplementedError("write your pallas kernel here")


# speedup vs baseline: 1.2690x; 1.2690x over previous
import functools

import jax
import jax.numpy as jnp
from jax import lax
from jax.experimental import pallas as pl
from jax.experimental.pallas import tpu as pltpu
from jax.experimental.pallas import tpu_sc as plsc

N = 10000
E = 320000
D = 128
MSG = 2 * D + 6
MSGP = 288
H = MSGP // 2
NC = 2
NS = 16
EPT = E // NS
B = 80
NBLK = EPT // B
NP = 10240
TROWS = NP // NS
ZROWS = 128

BN = 1000
BE = 4000


def _pre_node_body(h_ref, wd_ref, ws_ref, gd_ref, gs_ref):
    hb = h_ref[...]
    gd_ref[...] = lax.dot_general(hb, wd_ref[0], (((1,), (1,)), ((), ())),
                                  preferred_element_type=jnp.float32)
    gs_ref[...] = lax.dot_general(hb, ws_ref[0], (((1,), (1,)), ((), ())),
                                  preferred_element_type=jnp.float32)


def _pre_edge_body(e_ref, we_ref, ee_ref):
    ee_ref[...] = lax.dot_general(e_ref[...], we_ref[0], (((1,), (1,)), ((), ())),
                                  preferred_element_type=jnp.float32)


def _sc_body(gd_hbm, gs_hbm, ee_hbm, src_hbm, dst_hbm, s2_hbm,
             dst_v, srcg_v, dstg_v, gd_v, gs_v, ee_v, acc, sem1, sem2):
    c = lax.axis_index("c")
    s = lax.axis_index("s")
    coff = c * N

    def zrow(j, _):
        for k in range(H // 16):
            ee_v[j, pl.ds(16 * k, 16)] = jnp.zeros((16,), jnp.float32)
        return 0
    lax.fori_loop(0, B, zrow, 0)
    for k in range(TROWS // B):
        pltpu.sync_copy(ee_v, acc.at[pl.ds(s * TROWS + k * B, B)])
    plsc.subcore_barrier()

    def block(b, _):
        i0 = s * EPT + b * B
        pltpu.sync_copy(dst_hbm.at[pl.ds(i0, B)], dst_v)
        pltpu.sync_copy(src_hbm.at[pl.ds(i0, B)], srcg_v)
        offv = jnp.full((16,), coff, jnp.int32)
        for k in range(B // 16):
            sl = pl.ds(16 * k, 16)
            dstg_v[sl] = dst_v[sl] + offv
            srcg_v[sl] = srcg_v[sl] + offv
        cp1 = pltpu.async_copy(gd_hbm.at[dstg_v], gd_v, sem1)
        cp2 = pltpu.async_copy(gs_hbm.at[srcg_v], gs_v, sem2)
        pltpu.sync_copy(ee_hbm.at[pl.ds(c * E + i0, B)], ee_v)
        cp1.wait()
        cp2.wait()

        def row(j, _):
            for k in range(H // 16):
                sl = pl.ds(16 * k, 16)
                x = gd_v[j, sl] + gs_v[j, sl] + ee_v[j, sl]
                gd_v[j, sl] = jnp.maximum(x, 0.0)
            return 0
        lax.fori_loop(0, B, row, 0)
        pltpu.sync_copy(gd_v, acc.at[dst_v], add=True)
        return 0

    lax.fori_loop(0, NBLK, block, 0)
    plsc.subcore_barrier()
    for k in range(TROWS // ZROWS):
        r0 = s * TROWS + k * ZROWS
        pltpu.sync_copy(acc.at[pl.ds(r0, ZROWS)],
                        s2_hbm.at[pl.ds(c * NP + r0, ZROWS)])


def _post_body(s0_ref, s1_ref, h_ref, sn_ref, w20_ref, w21_ref,
               wih_ref, whh_ref, g_ref, b_ref, out_ref):
    dn = (((1,), (1,)), ((), ()))
    a_v = (lax.dot_general(s0_ref[...], w20_ref[...], dn,
                           preferred_element_type=jnp.float32) +
           lax.dot_general(s1_ref[...], w21_ref[...], dn,
                           preferred_element_type=jnp.float32))
    gi = lax.dot_general(a_v, wih_ref[...], dn,
                         preferred_element_type=jnp.float32)
    hb = h_ref[...]
    gh = lax.dot_general(hb, whh_ref[...], dn,
                         preferred_element_type=jnp.float32)
    r = jax.nn.sigmoid(gi[:, :D] + gh[:, :D])
    z = jax.nn.sigmoid(gi[:, D:2 * D] + gh[:, D:2 * D])
    n = jnp.tanh(gi[:, 2 * D:] + r * gh[:, 2 * D:])
    h_new = (1.0 - z) * n + z * hb
    h_new = h_new * sn_ref[...]
    mu = jnp.mean(h_new, axis=-1, keepdims=True)
    var = jnp.mean((h_new - mu) ** 2, axis=-1, keepdims=True)
    h_new = (h_new - mu) * lax.rsqrt(var + 1e-5) * g_ref[...] + b_ref[...]
    out_ref[...] = jnp.maximum(h_new, 0.0)


def kernel(h, edge_index, e, snorm_n, snorm_e, W_e1, W_e2, W_ih, W_hh,
           ln_gamma, ln_beta):
    src = edge_index[0]
    dst = edge_index[1]

    W1p = jnp.pad(W_e1, ((0, MSGP - MSG), (0, 0)))
    W1d = W1p[:, :D].reshape(NC, H, D)
    W1s = W1p[:, D:2 * D].reshape(NC, H, D)
    W1e = jnp.pad(W1p[:, 2 * D:], ((0, 0), (0, 2))).reshape(NC, H, 8)
    ep = jnp.pad(e, ((0, 0), (0, 2)))
    W2p = jnp.pad(W_e2, ((0, 0), (0, MSGP - MSG)))
    W20 = W2p[:, :H]
    W21 = W2p[:, H:]

    gd2, gs2 = pl.pallas_call(
        _pre_node_body,
        grid=(NC, N // BN),
        in_specs=[
            pl.BlockSpec((BN, D), lambda c, nb: (nb, 0)),
            pl.BlockSpec((1, H, D), lambda c, nb: (c, 0, 0)),
            pl.BlockSpec((1, H, D), lambda c, nb: (c, 0, 0)),
        ],
        out_specs=[
            pl.BlockSpec((BN, H), lambda c, nb: (c * (N // BN) + nb, 0)),
            pl.BlockSpec((BN, H), lambda c, nb: (c * (N // BN) + nb, 0)),
        ],
        out_shape=[
            jax.ShapeDtypeStruct((NC * N, H), jnp.float32),
            jax.ShapeDtypeStruct((NC * N, H), jnp.float32),
        ],
    )(h, W1d, W1s)

    ee2 = pl.pallas_call(
        _pre_edge_body,
        grid=(NC, E // BE),
        in_specs=[
            pl.BlockSpec((BE, 8), lambda c, eb: (eb, 0)),
            pl.BlockSpec((1, H, 8), lambda c, eb: (c, 0, 0)),
        ],
        out_specs=pl.BlockSpec((BE, H), lambda c, eb: (c * (E // BE) + eb, 0)),
        out_shape=jax.ShapeDtypeStruct((NC * E, H), jnp.float32),
    )(ep, W1e)

    mesh = plsc.VectorSubcoreMesh(core_axis_name="c", subcore_axis_name="s")
    s2 = pl.kernel(
        _sc_body,
        out_type=jax.ShapeDtypeStruct((NC * NP, H), jnp.float32),
        mesh=mesh,
        scratch_types=[
            pltpu.VMEM((B,), jnp.int32),
            pltpu.VMEM((B,), jnp.int32),
            pltpu.VMEM((B,), jnp.int32),
            pltpu.VMEM((B, H), jnp.float32),
            pltpu.VMEM((B, H), jnp.float32),
            pltpu.VMEM((B, H), jnp.float32),
            pltpu.VMEM_SHARED((NP, H), jnp.float32),
            pltpu.SemaphoreType.DMA,
            pltpu.SemaphoreType.DMA,
        ],
        compiler_params=pltpu.CompilerParams(use_tc_tiling_on_sc=False),
    )(gd2, gs2, ee2, src, dst)
    s0 = s2[:N]
    s1 = s2[NP:NP + N]

    h_new = pl.pallas_call(
        _post_body,
        grid=(N // BN,),
        in_specs=[
            pl.BlockSpec((BN, H), lambda nb: (nb, 0)),
            pl.BlockSpec((BN, H), lambda nb: (nb, 0)),
            pl.BlockSpec((BN, D), lambda nb: (nb, 0)),
            pl.BlockSpec((BN, 1), lambda nb: (nb, 0)),
            pl.BlockSpec((D, H), lambda nb: (0, 0)),
            pl.BlockSpec((D, H), lambda nb: (0, 0)),
            pl.BlockSpec((3 * D, D), lambda nb: (0, 0)),
            pl.BlockSpec((3 * D, D), lambda nb: (0, 0)),
            pl.BlockSpec((1, D), lambda nb: (0, 0)),
            pl.BlockSpec((1, D), lambda nb: (0, 0)),
        ],
        out_specs=pl.BlockSpec((BN, D), lambda nb: (nb, 0)),
        out_shape=jax.ShapeDtypeStruct((N, D), jnp.float32),
    )(s0, s1, h, snorm_n, W20, W21, W_ih, W_hh,
      ln_gamma.reshape(1, D), ln_beta.reshape(1, D))

    return (h_new, e)
